# Initial kernel scaffold; baseline (speedup 1.0000x reference)
import jax
import jax.numpy as jnp
from jax.experimental import pallas as pl


def kernel(x, y):
    raise NotImplementedError("write your pallas kernel here")


# speedup vs baseline: 1.8546x; 1.8546x over previous
import functools

import jax
import jax.numpy as jnp
from jax import lax
from jax.experimental import pallas as pl
from jax.experimental.pallas import tpu as pltpu
from jax.experimental.pallas import tpu_sc as plsc

_GAMMA = 1.0
_BIG = 1e10
_B = 16
_N = 512
_D = 64
_L = 2 * _N - 1
_LP = 1024
_WLEN = 528
_CH = 64
_NCH = _LP // _CH


def _skew_body(x_ref, y_ref, mm_ref):
    xb = x_ref[0]
    yb = y_ref[0]
    xn = jnp.sum(xb * xb, axis=1)
    yn = jnp.sum(yb * yb, axis=1)
    g = lax.dot_general(yb, xb, (((1,), (1,)), ((), ())),
                        preferred_element_type=jnp.float32)
    p = yn[:, None] + xn[None, :] - 2.0 * g
    p = jnp.clip(p, 0.0, None)
    p = jnp.concatenate(
        [p, jnp.full((_LP - _N, _N), _BIG, jnp.float32)], axis=0)
    col = lax.broadcasted_iota(jnp.int32, (_LP, _N), 1)
    for k in range(9):
        s = 1 << k
        rolled = jnp.concatenate([p[_LP - s:], p[:_LP - s]], axis=0)
        p = jnp.where((col & s) != 0, rolled, p)
    mm_ref[0] = p


def _skewed_distances(x, y):
    return pl.pallas_call(
        _skew_body,
        grid=(_B,),
        in_specs=[
            pl.BlockSpec((1, _N, _D), lambda b: (b, 0, 0)),
            pl.BlockSpec((1, _N, _D), lambda b: (b, 0, 0)),
        ],
        out_specs=pl.BlockSpec((1, _LP, _N), lambda b: (b, 0, 0)),
        out_shape=jax.ShapeDtypeStruct((_B, _LP, _N), jnp.float32),
    )(x, y)


def _log13(s):
    z = (s - 1.0) / (s + 1.0)
    z2 = z * z
    pol = jnp.full_like(s, 1.0 / 15.0)
    for q in (13.0, 11.0, 9.0, 7.0, 5.0, 3.0, 1.0):
        pol = pol * z2 + 1.0 / q
    return 2.0 * z * pol


def _sdtw_sc(mm):
    mesh = plsc.VectorSubcoreMesh(core_axis_name="c", subcore_axis_name="s")

    @functools.partial(
        pl.kernel,
        out_type=jax.ShapeDtypeStruct((_B, 16), jnp.float32),
        mesh=mesh,
        scratch_types=[
            pltpu.VMEM((_CH, _N), jnp.float32),
            pltpu.VMEM((_CH, _N), jnp.float32),
            pltpu.VMEM((3 * _WLEN,), jnp.float32),
            pltpu.VMEM((16,), jnp.float32),
            pltpu.SemaphoreType.DMA,
            pltpu.SemaphoreType.DMA,
        ],
    )
    def run(mm_hbm, out_hbm, buf0, buf1, wf, ovec, sem0, sem1):
        wid = lax.axis_index("c") * 16 + lax.axis_index("s")

        @pl.when(wid < _B)
        def _():
            b = wid
            bufs = (buf0, buf1)
            sems = (sem0, sem1)
            cps = [
                pltpu.make_async_copy(
                    mm_hbm.at[b, pl.ds(c * _CH, _CH)], bufs[c % 2],
                    sems[c % 2])
                for c in range(_NCH)
            ]
            cps[0].start()
            cps[0].wait()
            cps[1].start()

            big = jnp.full((16,), _BIG, jnp.float32)
            for j in range(3 * _WLEN // 16):
                wf[pl.ds(j * 16, 16)] = big
            mm00 = jnp.full((16,), buf0[0, pl.ds(0, 16)][0], jnp.float32)
            for j in range(_N // 16):
                off = j * 16
                wf[pl.ds(1 + off, 16)] = buf0[0, pl.ds(off, 16)]
                wf[pl.ds(_WLEN + 1 + off, 16)] = (
                    buf0[1, pl.ds(off, 16)] + mm00)

            def make_step(c, buf):
                def step(lloc, _):
                    l = c * _CH + lloc
                    b0 = (l % 3) * _WLEN
                    b1 = ((l - 1) % 3) * _WLEN
                    b2 = ((l + 1) % 3) * _WLEN
                    ilo = jnp.maximum(0, l - (_N - 1))
                    ihi = jnp.minimum(l, _N - 1)

                    def jbody(j, _):
                        off = j * 16
                        diag = wf[pl.ds(b2 + off, 16)]
                        up = wf[pl.ds(b1 + off, 16)]
                        left = wf[pl.ds(b1 + off + 1, 16)]
                        mmv = buf[lloc, pl.ds(off, 16)]
                        mn = jnp.minimum(diag, jnp.minimum(up, left))
                        ssum = (jnp.exp(mn - diag) + jnp.exp(mn - up)
                                + jnp.exp(mn - left))
                        val = mn - _GAMMA * _log13(ssum) + mmv
                        wf[pl.ds(b0 + off + 1, 16)] = val
                        return 0

                    lax.fori_loop(ilo // 16, ihi // 16 + 1, jbody, 0)
                    return 0
                return step

            for c in range(_NCH):
                lo = 2 if c == 0 else 0
                hi = _CH - 1 if c == _NCH - 1 else _CH
                lax.fori_loop(lo, hi, make_step(c, bufs[c % 2]), 0)
                if c + 1 < _NCH:
                    cps[c + 1].wait()
                    if c + 2 < _NCH:
                        cps[c + 2].start()

            ovec[...] = wf[pl.ds(2 * _WLEN + _N, 16)]
            pltpu.sync_copy(ovec, out_hbm.at[b])

    return run(mm)


def kernel(x, y):
    mm = _skewed_distances(x, y)
    out16 = _sdtw_sc(mm)
    return out16[:, 0]


# speedup vs baseline: 4.8667x; 2.6241x over previous
import functools

import jax
import jax.numpy as jnp
import numpy as np
from jax import lax
from jax.experimental import pallas as pl
from jax.experimental.pallas import tpu as pltpu
from jax.experimental.pallas import tpu_sc as plsc

_BIG = 1e10
_B = 16
_N = 512
_D = 64
_W = 528
_LP = 1056
_CH = 48
_NCH = _LP // _CH

_SARR = np.arange(_W)
_PV = 33 * (_SARR % 16) + _SARR // 16
_IVEC = np.clip(_PV - 1, 0, _N - 1).astype(np.int32)
_ANS_LANE = int(np.where(_PV == _N)[0][0])

_LOG1P = (-0.0008241868165887663, 0.008352398130590103, -0.037683667454471416,
          0.10260688142041036, -0.19815470327884796, 0.31584541595831483,
          -0.4967318828980979, 0.9997311430040153, 5.5390388619125375e-06)


def _softmin3(a, b, c):
    m1 = jnp.minimum(a, b)
    mx1 = jnp.maximum(a, b)
    mn = jnp.minimum(m1, c)
    mx2 = jnp.maximum(m1, c)
    u = jnp.exp(mn - mx2) + jnp.exp(mn - mx1)
    pol = jnp.full_like(u, _LOG1P[0])
    for q in _LOG1P[1:]:
        pol = pol * u + q
    return mn - pol


def _skew_body(xg_ref, y_ref, mm_ref):
    xb = xg_ref[0]
    yb = y_ref[0]
    xn = jnp.sum(xb * xb, axis=1)
    yn = jnp.sum(yb * yb, axis=1)
    g = lax.dot_general(yb, xb, (((1,), (1,)), ((), ())),
                        preferred_element_type=jnp.float32)
    p = yn[:, None] + xn[None, :] - 2.0 * g
    p = jnp.clip(p, 0.0, None)
    s2 = lax.broadcasted_iota(jnp.int32, (_N, _W), 1)
    pvv = 33 * (s2 & 15) + (s2 >> 4)
    p = jnp.where((pvv < 1) | (pvv > _N), _BIG, p)
    p = jnp.concatenate(
        [p, jnp.full((_LP - _N, _W), _BIG, jnp.float32)], axis=0)
    s3 = lax.broadcasted_iota(jnp.int32, (_LP, _W), 1)
    rv = 33 * (s3 & 15) + (s3 >> 4) - 1
    for k in range(9):
        sh = 1 << k
        rolled = jnp.concatenate([p[_LP - sh:], p[:_LP - sh]], axis=0)
        p = jnp.where((rv & sh) != 0, rolled, p)
    mm_ref[0] = p


def _skewed_distances(x, y):
    xg = jnp.take(x, jnp.asarray(_IVEC), axis=1)
    return pl.pallas_call(
        _skew_body,
        grid=(_B,),
        in_specs=[
            pl.BlockSpec((1, _W, _D), lambda b: (b, 0, 0)),
            pl.BlockSpec((1, _N, _D), lambda b: (b, 0, 0)),
        ],
        out_specs=pl.BlockSpec((1, _LP, _W), lambda b: (b, 0, 0)),
        out_shape=jax.ShapeDtypeStruct((_B, _LP, _W), jnp.float32),
    )(xg, y)


def _sdtw_sc(mm):
    mesh = plsc.VectorSubcoreMesh(core_axis_name="c", subcore_axis_name="s")

    @functools.partial(
        pl.kernel,
        out_type=jax.ShapeDtypeStruct((_B, 16), jnp.float32),
        mesh=mesh,
        scratch_types=[
            pltpu.VMEM((_CH, _W), jnp.float32),
            pltpu.VMEM((_CH, _W), jnp.float32),
            pltpu.VMEM((_W,), jnp.float32),
            pltpu.VMEM((_W,), jnp.float32),
            pltpu.VMEM((_W,), jnp.float32),
            pltpu.VMEM((16,), jnp.float32),
            pltpu.SemaphoreType.DMA,
            pltpu.SemaphoreType.DMA,
        ],
    )
    def run(mm_hbm, out_hbm, buf0, buf1, w0, w1, w2, ovec, sem0, sem1):
        wid = lax.axis_index("c") * 16 + lax.axis_index("s")

        @pl.when(wid < _B)
        def _():
            b = wid
            it = lax.iota(jnp.int32, 16)
            m0 = it == 0
            bigv = jnp.full((16,), _BIG, jnp.float32)

            def copy_chunk(c, buf, sem):
                return pltpu.make_async_copy(
                    mm_hbm.at[b, pl.ds(c * _CH, _CH)], buf, sem)

            def do_step(wn, wa, wb, buf, lloc):
                va = wa[pl.ds(_W - 17, 16)]
                vb = wb[pl.ds(_W - 17, 16)]
                preva = jnp.where(m0, bigv, va)
                prevb = jnp.where(m0, bigv, vb)
                left0 = wa[pl.ds(0, 16)]
                mm0 = buf[lloc, pl.ds(0, 16)]
                wn[pl.ds(0, 16)] = _softmin3(prevb, preva, left0) + mm0

                @plsc.parallel_loop(16, _W, step=16, unroll=4)
                def _(soff):
                    diag = wb[pl.ds(soff - 16, 16)]
                    up = wa[pl.ds(soff - 16, 16)]
                    left = wa[pl.ds(soff, 16)]
                    mmv = buf[lloc, pl.ds(soff, 16)]
                    wn[pl.ds(soff, 16)] = _softmin3(diag, up, left) + mmv

            def do_triples(base, ntrip, buf, coff):
                def triple(t, _):
                    lb = base + 3 * t
                    do_step(w0, w2, w1, buf, lb - coff)
                    do_step(w1, w0, w2, buf, lb + 1 - coff)
                    do_step(w2, w1, w0, buf, lb + 2 - coff)
                    return 0
                lax.fori_loop(0, ntrip, triple, 0)

            cp0 = copy_chunk(0, buf0, sem0)
            cp0.start()
            cp0.wait()
            copy_chunk(1, buf1, sem1).start()

            mm00 = jnp.full((16,), buf0[0, pl.ds(16, 16)][0], jnp.float32)
            for j in range(_W // 16):
                off = j * 16
                w0[pl.ds(off, 16)] = buf0[0, pl.ds(off, 16)]
                w1[pl.ds(off, 16)] = buf0[1, pl.ds(off, 16)] + mm00
            do_step(w2, w1, w0, buf0, 2)
            do_triples(3, (_CH - 3) // 3, buf0, 0)
            copy_chunk(2, buf0, sem0).start()
            cp1w = copy_chunk(1, buf1, sem1)
            cp1w.wait()

            def pair(t, _):
                c = 2 * t + 1
                do_triples(c * _CH, jnp.where(c == _NCH - 1, 5, _CH // 3),
                           buf1, c * _CH)

                @pl.when(c + 2 < _NCH)
                def _():
                    copy_chunk(c + 2, buf1, sem1).start()

                @pl.when(c + 1 < _NCH)
                def _():
                    copy_chunk(c + 1, buf0, sem0).wait()
                    do_triples((c + 1) * _CH, _CH // 3, buf0, (c + 1) * _CH)

                @pl.when(c + 3 < _NCH)
                def _():
                    copy_chunk(c + 3, buf0, sem0).start()

                @pl.when(c + 2 < _NCH)
                def _():
                    copy_chunk(c + 2, buf1, sem1).wait()
                return 0

            lax.fori_loop(0, _NCH // 2, pair, 0)

            ovec[...] = w2[pl.ds(_ANS_LANE - 15, 16)]
            pltpu.sync_copy(ovec, out_hbm.at[b])

    return run(mm)


def kernel(x, y):
    mm = _skewed_distances(x, y)
    out16 = _sdtw_sc(mm)
    return out16[:, 15]


# speedup vs baseline: 5.9380x; 1.2202x over previous
import functools

import jax
import jax.numpy as jnp
import numpy as np
from jax import lax
from jax.experimental import pallas as pl
from jax.experimental.pallas import tpu as pltpu
from jax.experimental.pallas import tpu_sc as plsc

_BIG = 1e10
_B = 16
_N = 512
_D = 64
_W = 528
_LP = 1056
_CH = 48
_NCH = _LP // _CH

_SARR = np.arange(_W)
_PV = 33 * (_SARR % 16) + _SARR // 16
_IVEC = np.clip(_PV - 1, 0, _N - 1).astype(np.int32)
_ANS_LANE = int(np.where(_PV == _N)[0][0])

_LOG1P = (0.008592109931055492, -0.06303373373867692, 0.2067238479723716,
          -0.4512964175334485, 0.9917296877716534, 0.0003529662470068695)


def _softmin3(a, b, c):
    m1 = jnp.minimum(a, b)
    mx1 = jnp.maximum(a, b)
    mn = jnp.minimum(m1, c)
    mx2 = jnp.maximum(m1, c)
    u = jnp.exp(mn - mx2) + jnp.exp(mn - mx1)
    pol = jnp.full_like(u, _LOG1P[0])
    for q in _LOG1P[1:]:
        pol = pol * u + q
    return mn - pol


def _skew_body(xg_ref, y_ref, mm_ref):
    xb = xg_ref[0]
    yb = y_ref[0]
    xn = jnp.sum(xb * xb, axis=1)
    yn = jnp.sum(yb * yb, axis=1)
    g = lax.dot_general(yb, xb, (((1,), (1,)), ((), ())),
                        preferred_element_type=jnp.float32)
    p = yn[:, None] + xn[None, :] - 2.0 * g
    p = jnp.clip(p, 0.0, None)
    s2 = lax.broadcasted_iota(jnp.int32, (_N, _W), 1)
    pvv = 33 * (s2 & 15) + (s2 >> 4)
    p = jnp.where((pvv < 1) | (pvv > _N), _BIG, p)
    p = jnp.concatenate(
        [p, jnp.full((_LP - _N, _W), _BIG, jnp.float32)], axis=0)
    s3 = lax.broadcasted_iota(jnp.int32, (_LP, _W), 1)
    rv = 33 * (s3 & 15) + (s3 >> 4) - 1
    for k in range(9):
        sh = 1 << k
        rolled = jnp.concatenate([p[_LP - sh:], p[:_LP - sh]], axis=0)
        p = jnp.where((rv & sh) != 0, rolled, p)
    mm_ref[0] = p


def _skewed_distances(x, y):
    xg = jnp.take(x, jnp.asarray(_IVEC), axis=1)
    return pl.pallas_call(
        _skew_body,
        grid=(_B,),
        in_specs=[
            pl.BlockSpec((1, _W, _D), lambda b: (b, 0, 0)),
            pl.BlockSpec((1, _N, _D), lambda b: (b, 0, 0)),
        ],
        out_specs=pl.BlockSpec((1, _LP, _W), lambda b: (b, 0, 0)),
        out_shape=jax.ShapeDtypeStruct((_B, _LP, _W), jnp.float32),
    )(xg, y)


def _sdtw_sc(mm):
    mesh = plsc.VectorSubcoreMesh(core_axis_name="c", subcore_axis_name="s")

    @functools.partial(
        pl.kernel,
        out_type=jax.ShapeDtypeStruct((_B, 16), jnp.float32),
        mesh=mesh,
        scratch_types=[
            pltpu.VMEM((_CH, _W), jnp.float32),
            pltpu.VMEM((_CH, _W), jnp.float32),
            pltpu.VMEM((_W,), jnp.float32),
            pltpu.VMEM((_W,), jnp.float32),
            pltpu.VMEM((_W,), jnp.float32),
            pltpu.VMEM((16,), jnp.float32),
            pltpu.SemaphoreType.DMA,
            pltpu.SemaphoreType.DMA,
        ],
    )
    def run(mm_hbm, out_hbm, buf0, buf1, w0, w1, w2, ovec, sem0, sem1):
        wid = lax.axis_index("c") * 16 + lax.axis_index("s")

        @pl.when(wid < _B)
        def _():
            b = wid
            it = lax.iota(jnp.int32, 16)
            m0 = it == 0
            bigv = jnp.full((16,), _BIG, jnp.float32)

            def copy_chunk(c, buf, sem):
                return pltpu.make_async_copy(
                    mm_hbm.at[b, pl.ds(c * _CH, _CH)], buf, sem)

            def do_step(wn, wa, wb, buf, lloc):
                va = wa[pl.ds(_W - 17, 16)]
                vb = wb[pl.ds(_W - 17, 16)]
                preva = jnp.where(m0, bigv, va)
                prevb = jnp.where(m0, bigv, vb)
                left0 = wa[pl.ds(0, 16)]
                mm0 = buf[lloc, pl.ds(0, 16)]
                wn[pl.ds(0, 16)] = _softmin3(prevb, preva, left0) + mm0

                @plsc.parallel_loop(16, _W, step=16, unroll=8)
                def _(soff):
                    diag = wb[pl.ds(soff - 16, 16)]
                    up = wa[pl.ds(soff - 16, 16)]
                    left = wa[pl.ds(soff, 16)]
                    mmv = buf[lloc, pl.ds(soff, 16)]
                    wn[pl.ds(soff, 16)] = _softmin3(diag, up, left) + mmv

            def do_triples(base, ntrip, buf, coff):
                def triple(t, _):
                    lb = base + 3 * t
                    do_step(w0, w2, w1, buf, lb - coff)
                    do_step(w1, w0, w2, buf, lb + 1 - coff)
                    do_step(w2, w1, w0, buf, lb + 2 - coff)
                    return 0
                lax.fori_loop(0, ntrip, triple, 0)

            cp0 = copy_chunk(0, buf0, sem0)
            cp0.start()
            cp0.wait()
            copy_chunk(1, buf1, sem1).start()

            mm00 = jnp.full((16,), buf0[0, pl.ds(16, 16)][0], jnp.float32)
            for j in range(_W // 16):
                off = j * 16
                w0[pl.ds(off, 16)] = buf0[0, pl.ds(off, 16)]
                w1[pl.ds(off, 16)] = buf0[1, pl.ds(off, 16)] + mm00
            do_step(w2, w1, w0, buf0, 2)
            do_triples(3, (_CH - 3) // 3, buf0, 0)
            copy_chunk(2, buf0, sem0).start()
            cp1w = copy_chunk(1, buf1, sem1)
            cp1w.wait()

            def pair(t, _):
                c = 2 * t + 1
                do_triples(c * _CH, jnp.where(c == _NCH - 1, 5, _CH // 3),
                           buf1, c * _CH)

                @pl.when(c + 2 < _NCH)
                def _():
                    copy_chunk(c + 2, buf1, sem1).start()

                @pl.when(c + 1 < _NCH)
                def _():
                    copy_chunk(c + 1, buf0, sem0).wait()
                    do_triples((c + 1) * _CH, _CH // 3, buf0, (c + 1) * _CH)

                @pl.when(c + 3 < _NCH)
                def _():
                    copy_chunk(c + 3, buf0, sem0).start()

                @pl.when(c + 2 < _NCH)
                def _():
                    copy_chunk(c + 2, buf1, sem1).wait()
                return 0

            lax.fori_loop(0, _NCH // 2, pair, 0)

            ovec[...] = w2[pl.ds(_ANS_LANE - 15, 16)]
            pltpu.sync_copy(ovec, out_hbm.at[b])

    return run(mm)


def kernel(x, y):
    mm = _skewed_distances(x, y)
    out16 = _sdtw_sc(mm)
    return out16[:, 15]
